# baseline (speedup 1.0000x reference)
import jax
import jax.numpy as jnp
from jax import lax
from jax.experimental import pallas as pl
from jax.experimental.pallas import tpu as pltpu
from jax.experimental.pallas import tpu_sc as plsc

N = 10000
D = 128
DH = D // 2
E = 320000
NC = 2
NS = 16
CH = 128
BASE = 160
E_PAD = NS * BASE * CH
NCHUNK = E_PAD // CH
N_PAD = 10240
RPT = N_PAD // NS
ZROWS = 32

BCH = 1
G = BASE // BCH
NB = 4
L = 3
SR = BCH * CH
NROW = E_PAD // SR
HRPT = N_PAD // 16 // NS

_MESH = plsc.VectorSubcoreMesh(
    core_axis_name="c", subcore_axis_name="s", num_cores=NC, num_subcores=NS
)


def _sc_body(with_deg):
    def body(x_hbm, src_hbm, dst_hbm, *rest):
        rest = list(rest)
        agg_out = rest.pop(0)
        if with_deg:
            deg_out = rest.pop(0)
        src_idx = rest.pop(0)
        dst_idx = rest.pop(0)
        bufs = [rest.pop(0) for _ in range(NB)]
        zbuf = rest.pop(0)
        if with_deg:
            hist = rest.pop(0)
            degstage = rest.pop(0)
            iidx = rest.pop(0)
        agg_sh = rest.pop(0)
        if with_deg:
            deg_sh = rest.pop(0)
        gsem = [rest.pop(0) for _ in range(NB)]
        ssem = [rest.pop(0) for _ in range(NB)]

        c = lax.axis_index("c")
        s = lax.axis_index("s")
        base_row = s * RPT
        ones_v = jnp.ones((16,), jnp.float32)

        def zrow(i, _):
            for k in range(DH // 16):
                zbuf[i, pl.ds(k * 16, 16)] = jnp.zeros((16,), jnp.float32)
            return 0

        lax.fori_loop(0, ZROWS, zrow, 0)
        for p in range(RPT // ZROWS):
            pltpu.sync_copy(zbuf, agg_sh.at[pl.ds(base_row + p * ZROWS, ZROWS)])

        if with_deg:
            def zh(i, _):
                hist[i] = jnp.zeros((16,), jnp.float32)
                return 0

            lax.fori_loop(0, N_PAD // 16, zh, 0)

            def zds(i, _):
                degstage[i] = jnp.zeros((16,), jnp.float32)
                return 0

            lax.fori_loop(0, HRPT, zds, 0)
            pltpu.sync_copy(degstage, deg_sh.at[pl.ds(s * HRPT, HRPT)])
            for r in range(N_PAD // 16 // CH):
                for k in range(CH // 16):
                    iidx[r, pl.ds(k * 16, 16)] = (
                        lax.iota(jnp.int32, 16) + (r * CH + k * 16))

        plsc.subcore_barrier()

        lo = s * G
        pltpu.sync_copy(src_hbm.at[pl.ds(lo, G)], src_idx)
        pltpu.sync_copy(dst_hbm.at[pl.ds(lo, G)], dst_idx)
        off = c * N

        def offrow(i, _):
            for k in range(SR // 16):
                v = src_idx[i, pl.ds(k * 16, 16)]
                src_idx[i, pl.ds(k * 16, 16)] = v + off
            return 0

        lax.fori_loop(0, G, offrow, 0)

        def gather_wait(b, j):
            pltpu.make_async_copy(x_hbm.at[src_idx.at[j]], bufs[b],
                                  gsem[b]).wait()

        def scatter_wait(b, j):
            pltpu.make_async_copy(bufs[b], agg_sh.at[dst_idx.at[j]],
                                  ssem[b]).wait()

        for b in range(L):
            pltpu.async_copy(x_hbm.at[src_idx.at[b]], bufs[b], gsem[b])

        def group(g, _):
            for b in range(NB):
                j = g * NB + b
                bn = (b + L) % NB
                gather_wait(b, j)
                pltpu.async_copy(bufs[b], agg_sh.at[dst_idx.at[j]], ssem[b],
                                 add=True)
                if with_deg:
                    @pl.when((b % 2) == c)
                    def _():
                        for k in range(SR // 16):
                            d16 = dst_idx[j, pl.ds(k * 16, 16)]
                            plsc.addupdate_scatter(
                                hist, [d16 >> 4, d16 & 15], ones_v)
                jn = j + L

                @pl.when(jn < G)
                def _():
                    @pl.when(j >= NB - L)
                    def _():
                        scatter_wait(bn, jn - NB)
                    pltpu.async_copy(x_hbm.at[src_idx.at[jn]], bufs[bn],
                                     gsem[bn])
            return 0

        lax.fori_loop(0, G // NB, group, 0)

        for b in range(NB):
            scatter_wait(b, G - NB + b)


        if with_deg:
            for r in range(N_PAD // 16 // CH):
                pltpu.sync_copy(hist.at[pl.ds(r * CH, CH)],
                                deg_sh.at[iidx.at[r]], add=True)

        plsc.subcore_barrier()

        pltpu.sync_copy(agg_sh.at[pl.ds(base_row, RPT)],
                        agg_out.at[c, pl.ds(base_row, RPT)])
        if with_deg:
            pltpu.sync_copy(deg_sh.at[pl.ds(s * HRPT, HRPT)],
                            deg_out.at[c, pl.ds(s * HRPT, HRPT)])

    return body


def _make_sc(with_deg):
    out_type = [jax.ShapeDtypeStruct((NC, N_PAD, DH), jnp.float32)]
    scratch = [
        pltpu.VMEM((G, SR), jnp.int32),
        pltpu.VMEM((G, SR), jnp.int32),
    ]
    scratch += [pltpu.VMEM((SR, DH), jnp.float32) for _ in range(NB)]
    if with_deg:
        out_type.append(
            jax.ShapeDtypeStruct((NC, N_PAD // 16, 16), jnp.float32))
    scratch.append(pltpu.VMEM((ZROWS, DH), jnp.float32))
    if with_deg:
        scratch.append(pltpu.VMEM((N_PAD // 16, 16), jnp.float32))
        scratch.append(pltpu.VMEM((HRPT, 16), jnp.float32))
        scratch.append(pltpu.VMEM((N_PAD // 16 // CH, CH), jnp.int32))
    scratch.append(pltpu.VMEM_SHARED((N_PAD, DH), jnp.float32))
    if with_deg:
        scratch.append(
            pltpu.VMEM_SHARED((N_PAD // 16, 16), jnp.float32))
    scratch += [pltpu.SemaphoreType.DMA for _ in range(2 * NB)]
    return pl.kernel(
        _sc_body(with_deg),
        out_type=out_type,
        mesh=_MESH,
        scratch_types=scratch,
        compiler_params=pltpu.CompilerParams(
            use_tc_tiling_on_sc=False, needs_layout_passes=False),
    )


_sc_agg_deg = _make_sc(True)
_sc_agg = _make_sc(False)

BLK = 2000


def _tc_body1(x_ref, a0, a1, d, ws, wn, b, out_ref):
    x = x_ref[...]
    agg = jnp.concatenate([a0[...], a1[...]], axis=1)
    deg = d[0] + d[1]
    hn = agg / jnp.maximum(deg, 1.0)
    y = jnp.dot(x, ws[...], preferred_element_type=jnp.float32)
    y = y + jnp.dot(hn, wn[...], preferred_element_type=jnp.float32)
    y = jnp.maximum(y + b[...], 0.0)
    out_ref[0] = y[:, :DH]
    out_ref[1] = y[:, DH:]


def _tc_body2(h_ref, a0, a1, d, ws, wn, b, out_ref):
    x = jnp.concatenate([h_ref[0], h_ref[1]], axis=1)
    agg = jnp.concatenate([a0[...], a1[...]], axis=1)
    deg = d[0] + d[1]
    hn = agg / jnp.maximum(deg, 1.0)
    y = jnp.dot(x, ws[...], preferred_element_type=jnp.float32)
    y = y + jnp.dot(hn, wn[...], preferred_element_type=jnp.float32)
    out_ref[...] = y + b[...]


_AGG_SPEC = pl.BlockSpec((BLK, DH), lambda i: (i, 0))
_DEG_SPEC = pl.BlockSpec((2, BLK, 1), lambda i: (0, i, 0))
_W_SPEC = pl.BlockSpec((D, D), lambda i: (0, 0))
_B_SPEC = pl.BlockSpec((1, D), lambda i: (0, 0))
_H2_SPEC = pl.BlockSpec((2, BLK, DH), lambda i: (0, i, 0))

_tc_fuse1 = pl.pallas_call(
    _tc_body1,
    grid=(N // BLK,),
    in_specs=[
        pl.BlockSpec((BLK, D), lambda i: (i, 0)),
        _AGG_SPEC, _AGG_SPEC, _DEG_SPEC, _W_SPEC, _W_SPEC, _B_SPEC,
    ],
    out_specs=_H2_SPEC,
    out_shape=jax.ShapeDtypeStruct((2, N, DH), jnp.float32),
)

_tc_fuse2 = pl.pallas_call(
    _tc_body2,
    grid=(N // BLK,),
    in_specs=[
        _H2_SPEC,
        _AGG_SPEC, _AGG_SPEC, _DEG_SPEC, _W_SPEC, _W_SPEC, _B_SPEC,
    ],
    out_specs=pl.BlockSpec((BLK, D), lambda i: (i, 0)),
    out_shape=jax.ShapeDtypeStruct((N, D), jnp.float32),
)


def kernel(in_feat, edge_index, W_self1, W_neigh1, b1, W_self2, W_neigh2, b2):
    ei = edge_index.astype(jnp.int32)
    pad = E_PAD - E
    src2 = jnp.concatenate(
        [ei[0], jnp.zeros((pad,), jnp.int32)]).reshape(NROW, SR)
    dst2d = jnp.concatenate(
        [ei[1], jnp.full((pad,), N, jnp.int32)]).reshape(NROW, SR)

    xcat = jnp.concatenate([in_feat[:, :DH], in_feat[:, DH:]], axis=0)
    agg1, degc = _sc_agg_deg(xcat, src2, dst2d)
    deg16 = degc.reshape(NC, N_PAD, 1)
    h2 = _tc_fuse1(in_feat, agg1[0], agg1[1], deg16,
                   W_self1, W_neigh1, b1.reshape(1, D))
    (agg2,) = _sc_agg(h2.reshape(2 * N, DH), src2, dst2d)
    out = _tc_fuse2(h2, agg2[0], agg2[1], deg16,
                    W_self2, W_neigh2, b2.reshape(1, D))
    return out


# speedup vs baseline: 1.6025x; 1.6025x over previous
import jax
import jax.numpy as jnp
from jax import lax
from jax.experimental import pallas as pl
from jax.experimental.pallas import tpu as pltpu
from jax.experimental.pallas import tpu_sc as plsc

N = 10000
D = 128
DH = D // 2
E = 320000
NC = 2
NS = 16
CH = 128
N_PAD = 10240
RPT = N_PAD // NS
ZROWS = 32

SR = 96
G = 210
NB = 5
L = 4
E_PAD = NS * G * SR
NROW = E_PAD // SR
HRPT = N_PAD // 16 // NS

_MESH = plsc.VectorSubcoreMesh(
    core_axis_name="c", subcore_axis_name="s", num_cores=NC, num_subcores=NS
)


def _sc_body(with_deg):
    def body(x_hbm, src_hbm, dst_hbm, *rest):
        rest = list(rest)
        agg_out = rest.pop(0)
        if with_deg:
            deg_out = rest.pop(0)
        src_idx = rest.pop(0)
        dst_idx = rest.pop(0)
        bufs = [rest.pop(0) for _ in range(NB)]
        zbuf = rest.pop(0)
        if with_deg:
            hist = rest.pop(0)
            degstage = rest.pop(0)
            iidx = rest.pop(0)
        agg_sh = rest.pop(0)
        if with_deg:
            deg_sh = rest.pop(0)
        gsem = [rest.pop(0) for _ in range(NB)]
        ssem = [rest.pop(0) for _ in range(NB)]

        c = lax.axis_index("c")
        s = lax.axis_index("s")
        base_row = s * RPT
        ones_v = jnp.ones((16,), jnp.float32)

        def zrow(i, _):
            for k in range(DH // 16):
                zbuf[i, pl.ds(k * 16, 16)] = jnp.zeros((16,), jnp.float32)
            return 0

        lax.fori_loop(0, ZROWS, zrow, 0)
        for p in range(RPT // ZROWS):
            pltpu.sync_copy(zbuf, agg_sh.at[pl.ds(base_row + p * ZROWS, ZROWS)])

        if with_deg:
            def zh(i, _):
                hist[i] = jnp.zeros((16,), jnp.float32)
                return 0

            lax.fori_loop(0, N_PAD // 16, zh, 0)

            def zds(i, _):
                degstage[i] = jnp.zeros((16,), jnp.float32)
                return 0

            lax.fori_loop(0, HRPT, zds, 0)
            pltpu.sync_copy(degstage, deg_sh.at[pl.ds(s * HRPT, HRPT)])
            for r in range(N_PAD // 16 // CH):
                for k in range(CH // 16):
                    iidx[r, pl.ds(k * 16, 16)] = (
                        lax.iota(jnp.int32, 16) + (r * CH + k * 16))

        plsc.subcore_barrier()

        lo = s * G
        pltpu.sync_copy(src_hbm.at[pl.ds(lo, G)], src_idx)
        pltpu.sync_copy(dst_hbm.at[pl.ds(lo, G)], dst_idx)
        off = c * N

        def offrow(i, _):
            for k in range(SR // 16):
                v = src_idx[i, pl.ds(k * 16, 16)]
                src_idx[i, pl.ds(k * 16, 16)] = v + off
            return 0

        lax.fori_loop(0, G, offrow, 0)

        def gather_wait(b, j):
            pltpu.make_async_copy(x_hbm.at[src_idx.at[j]], bufs[b],
                                  gsem[b]).wait()

        def scatter_wait(b, j):
            pltpu.make_async_copy(bufs[b], agg_sh.at[dst_idx.at[j]],
                                  ssem[b]).wait()

        for b in range(L):
            pltpu.async_copy(x_hbm.at[src_idx.at[b]], bufs[b], gsem[b])

        def group(g, _):
            for b in range(NB):
                j = g * NB + b
                bn = (b + L) % NB
                gather_wait(b, j)
                pltpu.async_copy(bufs[b], agg_sh.at[dst_idx.at[j]], ssem[b],
                                 add=True)
                if with_deg:
                    @pl.when((j % 2) == c)
                    def _():
                        for k in range(SR // 16):
                            d16 = dst_idx[j, pl.ds(k * 16, 16)]
                            plsc.addupdate_scatter(
                                hist, [d16 >> 4, d16 & 15], ones_v)
                jn = j + L

                @pl.when(jn < G)
                def _():
                    @pl.when(j >= NB - L)
                    def _():
                        scatter_wait(bn, jn - NB)
                    pltpu.async_copy(x_hbm.at[src_idx.at[jn]], bufs[bn],
                                     gsem[bn])
            return 0

        lax.fori_loop(0, G // NB, group, 0)

        for b in range(NB):
            scatter_wait(b, G - NB + b)


        if with_deg:
            for r in range(N_PAD // 16 // CH):
                pltpu.sync_copy(hist.at[pl.ds(r * CH, CH)],
                                deg_sh.at[iidx.at[r]], add=True)

        plsc.subcore_barrier()

        pltpu.sync_copy(agg_sh.at[pl.ds(base_row, RPT)],
                        agg_out.at[c, pl.ds(base_row, RPT)])
        if with_deg:
            pltpu.sync_copy(deg_sh.at[pl.ds(s * HRPT, HRPT)],
                            deg_out.at[c, pl.ds(s * HRPT, HRPT)])

    return body


def _make_sc(with_deg):
    out_type = [jax.ShapeDtypeStruct((NC, N_PAD, DH), jnp.float32)]
    scratch = [
        pltpu.VMEM((G, SR), jnp.int32),
        pltpu.VMEM((G, SR), jnp.int32),
    ]
    scratch += [pltpu.VMEM((SR, DH), jnp.float32) for _ in range(NB)]
    if with_deg:
        out_type.append(
            jax.ShapeDtypeStruct((NC, N_PAD // 16, 16), jnp.float32))
    scratch.append(pltpu.VMEM((ZROWS, DH), jnp.float32))
    if with_deg:
        scratch.append(pltpu.VMEM((N_PAD // 16, 16), jnp.float32))
        scratch.append(pltpu.VMEM((HRPT, 16), jnp.float32))
        scratch.append(pltpu.VMEM((N_PAD // 16 // CH, CH), jnp.int32))
    scratch.append(pltpu.VMEM_SHARED((N_PAD, DH), jnp.float32))
    if with_deg:
        scratch.append(
            pltpu.VMEM_SHARED((N_PAD // 16, 16), jnp.float32))
    scratch += [pltpu.SemaphoreType.DMA for _ in range(2 * NB)]
    return pl.kernel(
        _sc_body(with_deg),
        out_type=out_type,
        mesh=_MESH,
        scratch_types=scratch,
        compiler_params=pltpu.CompilerParams(
            use_tc_tiling_on_sc=False, needs_layout_passes=False),
    )


_sc_agg_deg = _make_sc(True)
_sc_agg = _make_sc(False)

BLK = 2000


def _tc_body1(x_ref, a0, a1, d, ws, wn, b, out_ref):
    x = x_ref[...]
    agg = jnp.concatenate([a0[...], a1[...]], axis=1)
    deg = d[0] + d[1]
    hn = agg / jnp.maximum(deg, 1.0)
    y = jnp.dot(x, ws[...], preferred_element_type=jnp.float32)
    y = y + jnp.dot(hn, wn[...], preferred_element_type=jnp.float32)
    y = jnp.maximum(y + b[...], 0.0)
    out_ref[0] = y[:, :DH]
    out_ref[1] = y[:, DH:]


def _tc_body2(h_ref, a0, a1, d, ws, wn, b, out_ref):
    x = jnp.concatenate([h_ref[0], h_ref[1]], axis=1)
    agg = jnp.concatenate([a0[...], a1[...]], axis=1)
    deg = d[0] + d[1]
    hn = agg / jnp.maximum(deg, 1.0)
    y = jnp.dot(x, ws[...], preferred_element_type=jnp.float32)
    y = y + jnp.dot(hn, wn[...], preferred_element_type=jnp.float32)
    out_ref[...] = y + b[...]


_AGG_SPEC = pl.BlockSpec((BLK, DH), lambda i: (i, 0))
_DEG_SPEC = pl.BlockSpec((2, BLK, 1), lambda i: (0, i, 0))
_W_SPEC = pl.BlockSpec((D, D), lambda i: (0, 0))
_B_SPEC = pl.BlockSpec((1, D), lambda i: (0, 0))
_H2_SPEC = pl.BlockSpec((2, BLK, DH), lambda i: (0, i, 0))

_tc_fuse1 = pl.pallas_call(
    _tc_body1,
    grid=(N // BLK,),
    in_specs=[
        pl.BlockSpec((BLK, D), lambda i: (i, 0)),
        _AGG_SPEC, _AGG_SPEC, _DEG_SPEC, _W_SPEC, _W_SPEC, _B_SPEC,
    ],
    out_specs=_H2_SPEC,
    out_shape=jax.ShapeDtypeStruct((2, N, DH), jnp.float32),
)

_tc_fuse2 = pl.pallas_call(
    _tc_body2,
    grid=(N // BLK,),
    in_specs=[
        _H2_SPEC,
        _AGG_SPEC, _AGG_SPEC, _DEG_SPEC, _W_SPEC, _W_SPEC, _B_SPEC,
    ],
    out_specs=pl.BlockSpec((BLK, D), lambda i: (i, 0)),
    out_shape=jax.ShapeDtypeStruct((N, D), jnp.float32),
)


def kernel(in_feat, edge_index, W_self1, W_neigh1, b1, W_self2, W_neigh2, b2):
    ei = edge_index.astype(jnp.int32)
    pad = E_PAD - E
    src2 = jnp.concatenate(
        [ei[0], jnp.zeros((pad,), jnp.int32)]).reshape(NROW, SR)
    dst2d = jnp.concatenate(
        [ei[1], jnp.full((pad,), N, jnp.int32)]).reshape(NROW, SR)

    xcat = jnp.concatenate([in_feat[:, :DH], in_feat[:, DH:]], axis=0)
    agg1, degc = _sc_agg_deg(xcat, src2, dst2d)
    deg16 = degc.reshape(NC, N_PAD, 1)
    h2 = _tc_fuse1(in_feat, agg1[0], agg1[1], deg16,
                   W_self1, W_neigh1, b1.reshape(1, D))
    (agg2,) = _sc_agg(h2.reshape(2 * N, DH), src2, dst2d)
    out = _tc_fuse2(h2, agg2[0], agg2[1], deg16,
                    W_self2, W_neigh2, b2.reshape(1, D))
    return out


# speedup vs baseline: 2.2432x; 1.3998x over previous
import jax
import jax.numpy as jnp
from jax import lax
from jax.experimental import pallas as pl
from jax.experimental.pallas import tpu as pltpu
from jax.experimental.pallas import tpu_sc as plsc

N = 10000
D = 128
DH = D // 2
E = 320000
NC = 2
NS = 16
CH = 128
N_PAD = 10240
RPT = N_PAD // NS
ZROWS = 32

SR = 80
G = 250
NB = 5
L = 4
E_PAD = NS * G * SR
NROW = E_PAD // SR
HRPT = N_PAD // 16 // NS

_MESH = plsc.VectorSubcoreMesh(
    core_axis_name="c", subcore_axis_name="s", num_cores=NC, num_subcores=NS
)


def _sc_body(with_deg):
    def body(x_hbm, src_hbm, dst_hbm, *rest):
        rest = list(rest)
        agg_out = rest.pop(0)
        if with_deg:
            deg_out = rest.pop(0)
        src_idx = rest.pop(0)
        dst_idx = rest.pop(0)
        bufs = [rest.pop(0) for _ in range(NB)]
        zbuf = rest.pop(0)
        if with_deg:
            hist = rest.pop(0)
            degstage = rest.pop(0)
            iidx = rest.pop(0)
        agg_sh = rest.pop(0)
        if with_deg:
            deg_sh = rest.pop(0)
        gsem = [rest.pop(0) for _ in range(NB)]
        ssem = [rest.pop(0) for _ in range(NB)]

        c = lax.axis_index("c")
        s = lax.axis_index("s")
        base_row = s * RPT
        ones_v = jnp.ones((16,), jnp.float32)

        def zrow(i, _):
            for k in range(DH // 16):
                zbuf[i, pl.ds(k * 16, 16)] = jnp.zeros((16,), jnp.float32)
            return 0

        lax.fori_loop(0, ZROWS, zrow, 0)
        for p in range(RPT // ZROWS):
            pltpu.sync_copy(zbuf, agg_sh.at[pl.ds(base_row + p * ZROWS, ZROWS)])

        if with_deg:
            def zh(i, _):
                hist[i] = jnp.zeros((16,), jnp.float32)
                return 0

            lax.fori_loop(0, N_PAD // 16, zh, 0)

            def zds(i, _):
                degstage[i] = jnp.zeros((16,), jnp.float32)
                return 0

            lax.fori_loop(0, HRPT, zds, 0)
            pltpu.sync_copy(degstage, deg_sh.at[pl.ds(s * HRPT, HRPT)])
            for r in range(N_PAD // 16 // CH):
                for k in range(CH // 16):
                    iidx[r, pl.ds(k * 16, 16)] = (
                        lax.iota(jnp.int32, 16) + (r * CH + k * 16))

        plsc.subcore_barrier()

        lo = s * G
        pltpu.sync_copy(src_hbm.at[pl.ds(lo, G)], src_idx)
        pltpu.sync_copy(dst_hbm.at[pl.ds(lo, G)], dst_idx)
        off = c * N

        def offrow(i, _):
            for k in range(SR // 16):
                v = src_idx[i, pl.ds(k * 16, 16)]
                src_idx[i, pl.ds(k * 16, 16)] = v + off
            return 0

        lax.fori_loop(0, G, offrow, 0)

        def gather_wait(b, j):
            pltpu.make_async_copy(x_hbm.at[src_idx.at[j]], bufs[b],
                                  gsem[b]).wait()

        def scatter_wait(b, j):
            pltpu.make_async_copy(bufs[b], agg_sh.at[dst_idx.at[j]],
                                  ssem[b]).wait()

        for b in range(L):
            pltpu.async_copy(x_hbm.at[src_idx.at[b]], bufs[b], gsem[b])

        def group(g, _):
            for b in range(NB):
                j = g * NB + b
                bn = (b + L) % NB
                gather_wait(b, j)
                pltpu.async_copy(bufs[b], agg_sh.at[dst_idx.at[j]], ssem[b],
                                 add=True)
                if with_deg:
                    @pl.when((j % 2) == c)
                    def _():
                        for k in range(SR // 16):
                            d16 = dst_idx[j, pl.ds(k * 16, 16)]
                            plsc.addupdate_scatter(
                                hist, [d16 >> 4, d16 & 15], ones_v)
                jn = j + L

                @pl.when(jn < G)
                def _():
                    @pl.when(j >= NB - L)
                    def _():
                        scatter_wait(bn, jn - NB)
                    pltpu.async_copy(x_hbm.at[src_idx.at[jn]], bufs[bn],
                                     gsem[bn])
            return 0

        lax.fori_loop(0, G // NB, group, 0)

        for b in range(NB):
            scatter_wait(b, G - NB + b)


        if with_deg:
            for r in range(N_PAD // 16 // CH):
                pltpu.sync_copy(hist.at[pl.ds(r * CH, CH)],
                                deg_sh.at[iidx.at[r]], add=True)

        plsc.subcore_barrier()

        pltpu.sync_copy(agg_sh.at[pl.ds(base_row, RPT)],
                        agg_out.at[c, pl.ds(base_row, RPT)])
        if with_deg:
            pltpu.sync_copy(deg_sh.at[pl.ds(s * HRPT, HRPT)],
                            deg_out.at[c, pl.ds(s * HRPT, HRPT)])

    return body


def _make_sc(with_deg):
    out_type = [jax.ShapeDtypeStruct((NC, N_PAD, DH), jnp.float32)]
    scratch = [
        pltpu.VMEM((G, SR), jnp.int32),
        pltpu.VMEM((G, SR), jnp.int32),
    ]
    scratch += [pltpu.VMEM((SR, DH), jnp.float32) for _ in range(NB)]
    if with_deg:
        out_type.append(
            jax.ShapeDtypeStruct((NC, N_PAD // 16, 16), jnp.float32))
    scratch.append(pltpu.VMEM((ZROWS, DH), jnp.float32))
    if with_deg:
        scratch.append(pltpu.VMEM((N_PAD // 16, 16), jnp.float32))
        scratch.append(pltpu.VMEM((HRPT, 16), jnp.float32))
        scratch.append(pltpu.VMEM((N_PAD // 16 // CH, CH), jnp.int32))
    scratch.append(pltpu.VMEM_SHARED((N_PAD, DH), jnp.float32))
    if with_deg:
        scratch.append(
            pltpu.VMEM_SHARED((N_PAD // 16, 16), jnp.float32))
    scratch += [pltpu.SemaphoreType.DMA for _ in range(2 * NB)]
    return pl.kernel(
        _sc_body(with_deg),
        out_type=out_type,
        mesh=_MESH,
        scratch_types=scratch,
        compiler_params=pltpu.CompilerParams(
            use_tc_tiling_on_sc=False, needs_layout_passes=False),
    )


_sc_agg_deg = _make_sc(True)
_sc_agg = _make_sc(False)

BLK = 2000


def _tc_body1(x_ref, a0, a1, d, ws, wn, b, out_ref):
    x = x_ref[...]
    agg = jnp.concatenate([a0[...], a1[...]], axis=1)
    deg = d[0] + d[1]
    hn = agg / jnp.maximum(deg, 1.0)
    y = jnp.dot(x, ws[...], preferred_element_type=jnp.float32)
    y = y + jnp.dot(hn, wn[...], preferred_element_type=jnp.float32)
    y = jnp.maximum(y + b[...], 0.0)
    out_ref[0] = y[:, :DH]
    out_ref[1] = y[:, DH:]


def _tc_body2(h_ref, a0, a1, d, ws, wn, b, out_ref):
    x = jnp.concatenate([h_ref[0], h_ref[1]], axis=1)
    agg = jnp.concatenate([a0[...], a1[...]], axis=1)
    deg = d[0] + d[1]
    hn = agg / jnp.maximum(deg, 1.0)
    y = jnp.dot(x, ws[...], preferred_element_type=jnp.float32)
    y = y + jnp.dot(hn, wn[...], preferred_element_type=jnp.float32)
    out_ref[...] = y + b[...]


_AGG_SPEC = pl.BlockSpec((BLK, DH), lambda i: (i, 0))
_DEG_SPEC = pl.BlockSpec((2, BLK, 1), lambda i: (0, i, 0))
_W_SPEC = pl.BlockSpec((D, D), lambda i: (0, 0))
_B_SPEC = pl.BlockSpec((1, D), lambda i: (0, 0))
_H2_SPEC = pl.BlockSpec((2, BLK, DH), lambda i: (0, i, 0))

_tc_fuse1 = pl.pallas_call(
    _tc_body1,
    grid=(N // BLK,),
    in_specs=[
        pl.BlockSpec((BLK, D), lambda i: (i, 0)),
        _AGG_SPEC, _AGG_SPEC, _DEG_SPEC, _W_SPEC, _W_SPEC, _B_SPEC,
    ],
    out_specs=_H2_SPEC,
    out_shape=jax.ShapeDtypeStruct((2, N, DH), jnp.float32),
)

_tc_fuse2 = pl.pallas_call(
    _tc_body2,
    grid=(N // BLK,),
    in_specs=[
        _H2_SPEC,
        _AGG_SPEC, _AGG_SPEC, _DEG_SPEC, _W_SPEC, _W_SPEC, _B_SPEC,
    ],
    out_specs=pl.BlockSpec((BLK, D), lambda i: (i, 0)),
    out_shape=jax.ShapeDtypeStruct((N, D), jnp.float32),
)


def kernel(in_feat, edge_index, W_self1, W_neigh1, b1, W_self2, W_neigh2, b2):
    ei = edge_index.astype(jnp.int32)
    pad = E_PAD - E
    src2 = jnp.concatenate(
        [ei[0], jnp.zeros((pad,), jnp.int32)]).reshape(NROW, SR)
    dst2d = jnp.concatenate(
        [ei[1], jnp.full((pad,), N, jnp.int32)]).reshape(NROW, SR)

    xcat = jnp.concatenate([in_feat[:, :DH], in_feat[:, DH:]], axis=0)
    agg1, degc = _sc_agg_deg(xcat, src2, dst2d)
    deg16 = degc.reshape(NC, N_PAD, 1)
    h2 = _tc_fuse1(in_feat, agg1[0], agg1[1], deg16,
                   W_self1, W_neigh1, b1.reshape(1, D))
    (agg2,) = _sc_agg(h2.reshape(2 * N, DH), src2, dst2d)
    out = _tc_fuse2(h2, agg2[0], agg2[1], deg16,
                    W_self2, W_neigh2, b2.reshape(1, D))
    return out
